# baseline (speedup 1.0000x reference)
import functools

import jax
import jax.numpy as jnp
from jax import lax
from jax.experimental import pallas as pl
from jax.experimental.pallas import tpu as pltpu
from jax.experimental.pallas import tpu_sc as plsc

AUDIO_TOKEN_ID = 7
B, S, D = 2, 4096, 2048
A = 1024
NC, NS, L = 2, 16, 16
NW = NC * NS
RPW = (B * S) // NW
WPB = NW // B
CH = 16
NCH = RPW // CH
SCH = L


def _body(emb_hbm, audio_hbm, tok_hbm, out_hbm,
          tok_v, pos_v, cbuf0, cbuf1, sbuf, didx_v, aidx_v,
          lsem0, lsem1, ssem0, ssem1, gsem, scsem):
    wid = lax.axis_index("s") * NC + lax.axis_index("c")
    batch = wid // WPB
    w_local = wid % WPB
    row0 = wid * RPW

    pltpu.sync_copy(tok_hbm.at[pl.ds(batch * S, S)], tok_v)

    zeros = jnp.zeros((L,), jnp.int32)

    def pbody(i, acc):
        t = tok_v[pl.ds(i * L, L)]
        return acc + (t == AUDIO_TOKEN_ID).astype(jnp.int32)

    acc = lax.fori_loop(0, w_local * (RPW // L), pbody, zeros)
    prefix = jnp.sum(acc)

    lane = lax.broadcasted_iota(jnp.int32, (L,), 0)
    base_local = w_local * RPW
    cnt = jnp.int32(0)
    for i in range(RPW // L):
        t = tok_v[pl.ds(base_local + i * L, L)]
        m = t == AUDIO_TOKEN_ID
        m32 = m.astype(jnp.int32)
        rank = plsc.cumsum(m32) - 1
        plsc.store_scatter(pos_v, [cnt + rank], row0 + i * L + lane, mask=m)
        cnt = cnt + jnp.sum(m32)

    copies = [
        pltpu.async_copy(
            emb_hbm.at[pl.ds(row0 + c * CH, CH)],
            out_hbm.at[pl.ds(row0 + c * CH, CH)],
            (lsem0, lsem1)[c % 2])
        for c in range(NCH)
    ]
    for h in copies:
        h.wait()

    nch = (cnt + SCH - 1) // SCH

    def sbody(c, carry):
        j = jnp.minimum(c * SCH + lane, cnt - 1)
        didx_v[...] = plsc.load_gather(pos_v, [j])
        aidx_v[...] = batch * A + jnp.minimum(prefix + j, A - 1)
        pltpu.async_copy(audio_hbm.at[aidx_v], sbuf, gsem).wait()
        pltpu.async_copy(sbuf, out_hbm.at[didx_v], scsem).wait()
        return carry

    lax.fori_loop(0, nch, sbody, jnp.int32(0))


@jax.jit
def _run(emb2d, audio2d, tok1d):
    mesh = plsc.VectorSubcoreMesh(core_axis_name="c", subcore_axis_name="s",
                                  num_cores=NC, num_subcores=NS)
    f = pl.kernel(
        _body,
        out_type=jax.ShapeDtypeStruct((B * S, D), jnp.float32),
        mesh=mesh,
        compiler_params=pltpu.CompilerParams(needs_layout_passes=False),
        scratch_types=[
            pltpu.VMEM((S,), jnp.int32),
            pltpu.VMEM((RPW,), jnp.int32),
            pltpu.VMEM((CH, D), jnp.float32),
            pltpu.VMEM((CH, D), jnp.float32),
            pltpu.VMEM((SCH, D), jnp.float32),
            pltpu.VMEM((L,), jnp.int32),
            pltpu.VMEM((L,), jnp.int32),
            pltpu.SemaphoreType.DMA,
            pltpu.SemaphoreType.DMA,
            pltpu.SemaphoreType.DMA,
            pltpu.SemaphoreType.DMA,
            pltpu.SemaphoreType.DMA,
            pltpu.SemaphoreType.DMA,
        ],
    )
    return f(emb2d, audio2d, tok1d)


def kernel(embeddings, audio_embeddings, token_ids):
    emb2d = embeddings.reshape(B * S, D)
    audio2d = audio_embeddings.reshape(B * A, D)
    tok1d = token_ids.reshape(B * S).astype(jnp.int32)
    out = _run(emb2d, audio2d, tok1d)
    return out.reshape(B, S, D)


# speedup vs baseline: 27.8428x; 27.8428x over previous
import functools

import jax
import jax.numpy as jnp
from jax import lax
from jax.experimental import pallas as pl
from jax.experimental.pallas import tpu as pltpu
from jax.experimental.pallas import tpu_sc as plsc

AUDIO_TOKEN_ID = 7
B, S, D = 2, 4096, 2048
A = 1024
NC, NS, L = 2, 16, 16
NW = NC * NS
RPW = (B * S) // NW
WPB = NW // B
CH = 16
NCH = RPW // CH
SCH = L


def _body(emb_hbm, audio_hbm, tok_hbm, out_hbm,
          tok_v, pos_v, cbuf0, cbuf1, sbuf, didx_v, aidx_v,
          lsem0, lsem1, ssem0, ssem1, gsem, scsem):
    wid = lax.axis_index("s") * NC + lax.axis_index("c")
    batch = wid // WPB
    w_local = wid % WPB
    row0 = wid * RPW

    pltpu.sync_copy(tok_hbm.at[pl.ds(batch * S, S)], tok_v)

    zeros = jnp.zeros((L,), jnp.int32)

    def pbody(i, acc):
        t = tok_v[pl.ds(i * L, L)]
        return acc + (t == AUDIO_TOKEN_ID).astype(jnp.int32)

    acc = lax.fori_loop(0, w_local * (RPW // L), pbody, zeros)
    prefix = jnp.sum(acc)

    lane = lax.broadcasted_iota(jnp.int32, (L,), 0)
    base_local = w_local * RPW
    cnt = jnp.int32(0)
    for i in range(RPW // L):
        t = tok_v[pl.ds(base_local + i * L, L)]
        m = t == AUDIO_TOKEN_ID
        m32 = m.astype(jnp.int32)
        rank = plsc.cumsum(m32) - 1
        plsc.store_scatter(pos_v, [cnt + rank], row0 + i * L + lane, mask=m)
        cnt = cnt + jnp.sum(m32)

    cbufs = (cbuf0, cbuf1)
    lsems = (lsem0, lsem1)
    ssems = (ssem0, ssem1)
    loads = [None] * NCH
    stores = [None] * NCH
    for c in range(NCH + 1):
        if c < NCH:
            bb = c % 2
            if c >= 2:
                stores[c - 2].wait()
            loads[c] = pltpu.async_copy(
                emb_hbm.at[pl.ds(row0 + c * CH, CH)], cbufs[bb], lsems[bb])
        if c >= 1:
            pb = (c - 1) % 2
            loads[c - 1].wait()
            stores[c - 1] = pltpu.async_copy(
                cbufs[pb], out_hbm.at[pl.ds(row0 + (c - 1) * CH, CH)],
                ssems[pb])
    stores[NCH - 2].wait()
    stores[NCH - 1].wait()

    nch = (cnt + SCH - 1) // SCH

    def sbody(c, carry):
        j = jnp.minimum(c * SCH + lane, cnt - 1)
        didx_v[...] = plsc.load_gather(pos_v, [j])
        aidx_v[...] = batch * A + jnp.minimum(prefix + j, A - 1)
        pltpu.async_copy(audio_hbm.at[aidx_v], sbuf, gsem).wait()
        pltpu.async_copy(sbuf, out_hbm.at[didx_v], scsem).wait()
        return carry

    lax.fori_loop(0, nch, sbody, jnp.int32(0))


@jax.jit
def _run(emb2d, audio2d, tok1d):
    mesh = plsc.VectorSubcoreMesh(core_axis_name="c", subcore_axis_name="s",
                                  num_cores=NC, num_subcores=NS)
    f = pl.kernel(
        _body,
        out_type=jax.ShapeDtypeStruct((B * S, D), jnp.float32),
        mesh=mesh,
        compiler_params=pltpu.CompilerParams(needs_layout_passes=False),
        scratch_types=[
            pltpu.VMEM((S,), jnp.int32),
            pltpu.VMEM((RPW,), jnp.int32),
            pltpu.VMEM((CH, D), jnp.float32),
            pltpu.VMEM((CH, D), jnp.float32),
            pltpu.VMEM((SCH, D), jnp.float32),
            pltpu.VMEM((L,), jnp.int32),
            pltpu.VMEM((L,), jnp.int32),
            pltpu.SemaphoreType.DMA,
            pltpu.SemaphoreType.DMA,
            pltpu.SemaphoreType.DMA,
            pltpu.SemaphoreType.DMA,
            pltpu.SemaphoreType.DMA,
            pltpu.SemaphoreType.DMA,
        ],
    )
    return f(emb2d, audio2d, tok1d)


def kernel(embeddings, audio_embeddings, token_ids):
    emb2d = embeddings.reshape(B * S, D)
    audio2d = audio_embeddings.reshape(B * A, D)
    tok1d = token_ids.reshape(B * S).astype(jnp.int32)
    out = _run(emb2d, audio2d, tok1d)
    return out.reshape(B, S, D)
